# baseline (speedup 1.0000x reference)
import functools

import jax
import jax.numpy as jnp
from jax import lax
from jax.experimental import pallas as pl
from jax.experimental.pallas import tpu as pltpu
from jax.experimental.pallas import tpu_sc as plsc

N = 10000
E = 320000
H = 128
C = 40
BN_EPS = 1e-5

NS = 16

EB = 128
EROWS = E // EB
CH_BIG = (24, 24, 24, 24, 24, 24, 16)
CH_MAIN15 = (24, 24, 24, 8)
TAILR = 24
RPT_BIG_E = sum(CH_BIG)
ICH = 24
RPT_BIG = 640
RPT_LAST = N - (NS - 1) * RPT_BIG
ZROWS = 16
WCH = 80

_sc_mesh = plsc.VectorSubcoreMesh(
    core_axis_name="c", subcore_axis_name="s", num_cores=1, num_subcores=NS
)


@functools.partial(
    pl.kernel,
    out_type=jax.ShapeDtypeStruct((N, H), jnp.float32),
    mesh=_sc_mesh,
    scratch_types=[
        pltpu.VMEM((ICH, EB), jnp.int32),
        pltpu.VMEM((ICH, EB), jnp.int32),
        pltpu.VMEM((ICH, EB), jnp.int32),
        pltpu.VMEM((ICH, EB), jnp.int32),
        pltpu.VMEM((EB, H), jnp.float32),
        pltpu.VMEM((EB, H), jnp.float32),
        pltpu.VMEM((ZROWS, H), jnp.float32),
        pltpu.VMEM_SHARED((N + EB, H), jnp.float32),
        pltpu.SemaphoreType.DMA,
        pltpu.SemaphoreType.DMA,
        pltpu.SemaphoreType.DMA,
        pltpu.SemaphoreType.DMA,
    ],
)
def _sc_agg(x_hbm, e_hbm, tail_hbm, out_hbm, sidx0, didx0, sidx1, didx1,
            rows0, rows1, zbuf, acc, gs0, gs1, is0, is1):
    s = lax.axis_index("s")

    with jax.named_scope("agg_init"):
        def zfill(i, _):
            def zrow(j, _):
                zbuf[i, pl.ds(j * 16, 16)] = jnp.zeros((16,), jnp.float32)
                return 0
            return lax.fori_loop(0, H // 16, zrow, 0)
        lax.fori_loop(0, ZROWS, zfill, 0)

        rbase = pl.multiple_of(s * RPT_BIG, 8)
        nrows = lax.select(s == NS - 1, RPT_LAST, RPT_BIG)
        def zcopy(i, _):
            pltpu.sync_copy(zbuf, acc.at[pl.ds(rbase + i * ZROWS, ZROWS)])
            return 0
        lax.fori_loop(0, nrows // ZROWS, zcopy, 0)

        @pl.when(s == NS - 1)
        def _():
            def ztrash(i, _):
                pltpu.sync_copy(zbuf, acc.at[pl.ds(N + i * ZROWS, ZROWS)])
                return 0
            lax.fori_loop(0, EB // ZROWS, ztrash, 0)

        plsc.subcore_barrier()

    def gather(sref, j, buf, sem):
        pltpu.async_copy(x_hbm.at[sref.at[j]], buf, sem)

    def gwait(sref, buf, sem):
        pltpu.make_async_copy(x_hbm.at[sref.at[0]], buf, sem).wait()

    def scatter(dref, j, buf):
        pltpu.sync_copy(buf, acc.at[dref.at[j]], add=True)

    _pairs = ((sidx0, didx0, is0), (sidx1, didx1, is1))

    def iload(k, ref, erow, ch):
        ps, pd, ise = _pairs[k % 2]
        pltpu.async_copy(ref.at[0, pl.ds(erow, ch)], ps.at[pl.ds(0, ch)], ise)
        pltpu.async_copy(ref.at[1, pl.ds(erow, ch)], pd.at[pl.ds(0, ch)], ise)

    def iwait(k, ref, erow, ch):
        ps, pd, ise = _pairs[k % 2]
        pltpu.make_async_copy(ref.at[0, pl.ds(erow, ch)],
                              ps.at[pl.ds(0, ch)], ise).wait()
        pltpu.make_async_copy(ref.at[1, pl.ds(erow, ch)],
                              pd.at[pl.ds(0, ch)], ise).wait()

    def run_chunk(k, ch):
        sref, dref, _ = _pairs[k % 2]
        gather(sref, 0, rows0, gs0)
        def body(g, _):
            gather(sref, 2 * g + 1, rows1, gs1)
            gwait(sref, rows0, gs0)
            scatter(dref, 2 * g, rows0)
            gather(sref, 2 * g + 2, rows0, gs0)
            gwait(sref, rows1, gs1)
            scatter(dref, 2 * g + 1, rows1)
            return 0
        lax.fori_loop(0, ch // 2 - 1, body, 0)
        gather(sref, ch - 1, rows1, gs1)
        gwait(sref, rows0, gs0)
        scatter(dref, ch - 2, rows0)
        gwait(sref, rows1, gs1)
        scatter(dref, ch - 1, rows1)

    def run_sched(sched):
        iload(0, *sched[0])
        for k, (ref, erow, ch) in enumerate(sched):
            if k + 1 < len(sched):
                iload(k + 1, *sched[k + 1])
            iwait(k, ref, erow, ch)
            run_chunk(k, ch)

    with jax.named_scope("agg_edges"):
        @pl.when(s < NS - 1)
        def _():
            sched, off = [], 0
            for ch in CH_BIG:
                sched.append((e_hbm, pl.multiple_of(s * RPT_BIG_E + off, 8), ch))
                off += ch
            run_sched(sched)

        @pl.when(s == NS - 1)
        def _():
            sched, off = [], 0
            for ch in CH_MAIN15:
                sched.append((e_hbm, (NS - 1) * RPT_BIG_E + off, ch))
                off += ch
            sched.append((tail_hbm, 0, TAILR))
            run_sched(sched)

    with jax.named_scope("agg_wb"):
        plsc.subcore_barrier()

        def wb(nch):
            ds_ = [pltpu.async_copy(
                acc.at[pl.ds(rbase + k * WCH, WCH)],
                out_hbm.at[pl.ds(rbase + k * WCH, WCH)], gs0)
                for k in range(nch)]
            for d in ds_:
                d.wait()

        @pl.when(s < NS - 1)
        def _():
            wb(RPT_BIG // WCH)

        @pl.when(s == NS - 1)
        def _():
            wb(RPT_LAST // WCH)


def _mlp_body(x_ref, agg_ref, w0, b0, w1, b1, w2, b2, out_ref):
    h = x_ref[...] + agg_ref[...]
    for w, b in ((w0, b0), (w1, b1), (w2, b2)):
        h = jnp.dot(h, w[...], preferred_element_type=jnp.float32)
        h = jnp.maximum(h + b[...], 0.0)
    out_ref[...] = h


def _head_body(x_ref, agg_ref, w0, b0, w1, b1, w2, b2, fcw, fcb, out_ref):
    h = x_ref[...] + agg_ref[...]
    for w, b in ((w0, b0), (w1, b1), (w2, b2)):
        h = jnp.dot(h, w[...], preferred_element_type=jnp.float32)
        h = jnp.maximum(h + b[...], 0.0)
    logits = jnp.dot(h, fcw[...], preferred_element_type=jnp.float32) + fcb[...]
    m = jnp.max(logits, axis=1, keepdims=True)
    z = logits - m
    lse = jnp.log(jnp.sum(jnp.exp(z), axis=1, keepdims=True))
    out_ref[...] = z - lse


_ROWS_BLK = 1000
_GRID = N // _ROWS_BLK

_x_spec = pl.BlockSpec((_ROWS_BLK, H), lambda i: (i, 0))
_w_spec = pl.BlockSpec((H, H), lambda i: (0, 0))
_b_spec = pl.BlockSpec((1, H), lambda i: (0, 0))


def _mlp_call(x, agg, w0, b0, w1, b1, w2, b2):
    return pl.pallas_call(
        _mlp_body,
        grid=(_GRID,),
        in_specs=[_x_spec, _x_spec,
                  _w_spec, _b_spec, _w_spec, _b_spec, _w_spec, _b_spec],
        out_specs=pl.BlockSpec((_ROWS_BLK, H), lambda i: (i, 0)),
        out_shape=jax.ShapeDtypeStruct((N, H), jnp.float32),
    )(x, agg, w0, b0, w1, b1, w2, b2)


def _head_call(x, agg, w0, b0, w1, b1, w2, b2, fcw, fcb):
    return pl.pallas_call(
        _head_body,
        grid=(_GRID,),
        in_specs=[_x_spec, _x_spec,
                  _w_spec, _b_spec, _w_spec, _b_spec, _w_spec, _b_spec,
                  pl.BlockSpec((H, C), lambda i: (0, 0)),
                  pl.BlockSpec((1, C), lambda i: (0, 0))],
        out_specs=pl.BlockSpec((_ROWS_BLK, C), lambda i: (i, 0)),
        out_shape=jax.ShapeDtypeStruct((N, C), jnp.float32),
    )(x, agg, w0, b0, w1, b1, w2, b2, fcw, fcb)


def _fold_bn(params, prefix):
    inv_std = 1.0 / jnp.sqrt(1.0 + BN_EPS)
    out = []
    for i in range(3):
        scale = params[f"{prefix}_g{i}"] * inv_std
        out.append(params[f"{prefix}_W{i}"] * scale[None, :])
        out.append((params[f"{prefix}_b{i}"] * scale
                    + params[f"{prefix}_beta{i}"])[None, :])
    return out


def kernel(x, edge_index, edge_attr, params):
    del edge_attr
    e3 = edge_index.astype(jnp.int32).reshape(2, EROWS, EB)
    npadr = TAILR - 20
    pad_dst = N + (jnp.arange(npadr * EB, dtype=jnp.int32) % EB)
    pad_block = jnp.stack([jnp.zeros((npadr * EB,), jnp.int32),
                           pad_dst]).reshape(2, npadr, EB)
    tail = jnp.concatenate([e3[:, EROWS - 20:], pad_block], axis=1)

    c1 = _fold_bn(params, "c1")
    c2 = _fold_bn(params, "c2")
    fcw = params["fc_W"]
    fcb = params["fc_b"][None, :]

    agg1 = _sc_agg(x, e3, tail)
    h1 = _mlp_call(x, agg1, *c1)
    agg2 = _sc_agg(h1, e3, tail)
    return _head_call(h1, agg2, *c2, fcw, fcb)


# speedup vs baseline: 1.0081x; 1.0081x over previous
import functools

import jax
import jax.numpy as jnp
from jax import lax
from jax.experimental import pallas as pl
from jax.experimental.pallas import tpu as pltpu
from jax.experimental.pallas import tpu_sc as plsc

N = 10000
E = 320000
H = 128
C = 40
BN_EPS = 1e-5

NS = 16

EB = 128
EROWS = E // EB
CH_BIG = (40, 40, 40, 40)
CH_MAIN15 = (40, 40)
TAILR = 24
RPT_BIG_E = sum(CH_BIG)
ICH = 40
RPT_BIG = 640
RPT_LAST = N - (NS - 1) * RPT_BIG
ZROWS = 16
WCH = 80

_sc_mesh = plsc.VectorSubcoreMesh(
    core_axis_name="c", subcore_axis_name="s", num_cores=1, num_subcores=NS
)


@functools.partial(
    pl.kernel,
    out_type=jax.ShapeDtypeStruct((N, H), jnp.float32),
    mesh=_sc_mesh,
    scratch_types=[
        pltpu.VMEM((ICH, EB), jnp.int32),
        pltpu.VMEM((ICH, EB), jnp.int32),
        pltpu.VMEM((EB, H), jnp.float32),
        pltpu.VMEM((EB, H), jnp.float32),
        pltpu.VMEM((ZROWS, H), jnp.float32),
        pltpu.VMEM_SHARED((N + EB, H), jnp.float32),
        pltpu.SemaphoreType.DMA,
        pltpu.SemaphoreType.DMA,
    ],
)
def _sc_agg(x_hbm, e_hbm, tail_hbm, out_hbm, sidx, didx, rows0, rows1,
            zbuf, acc, gs0, gs1):
    s = lax.axis_index("s")

    with jax.named_scope("agg_init"):
        def zfill(i, _):
            def zrow(j, _):
                zbuf[i, pl.ds(j * 16, 16)] = jnp.zeros((16,), jnp.float32)
                return 0
            return lax.fori_loop(0, H // 16, zrow, 0)
        lax.fori_loop(0, ZROWS, zfill, 0)

        rbase = pl.multiple_of(s * RPT_BIG, 8)
        nrows = lax.select(s == NS - 1, RPT_LAST, RPT_BIG)
        def zcopy(i, _):
            pltpu.sync_copy(zbuf, acc.at[pl.ds(rbase + i * ZROWS, ZROWS)])
            return 0
        lax.fori_loop(0, nrows // ZROWS, zcopy, 0)

        @pl.when(s == NS - 1)
        def _():
            def ztrash(i, _):
                pltpu.sync_copy(zbuf, acc.at[pl.ds(N + i * ZROWS, ZROWS)])
                return 0
            lax.fori_loop(0, EB // ZROWS, ztrash, 0)

        plsc.subcore_barrier()

    def gather(j, buf, sem):
        pltpu.async_copy(x_hbm.at[sidx.at[j]], buf, sem)

    def gwait(buf, sem):
        pltpu.make_async_copy(x_hbm.at[sidx.at[0]], buf, sem).wait()

    def scatter(j, buf):
        pltpu.sync_copy(buf, acc.at[didx.at[j]], add=True)

    def run_chunk(eref, erow, ch):
        pltpu.sync_copy(eref.at[0, pl.ds(erow, ch)], sidx.at[pl.ds(0, ch)])
        pltpu.sync_copy(eref.at[1, pl.ds(erow, ch)], didx.at[pl.ds(0, ch)])
        gather(0, rows0, gs0)
        def body(g, _):
            gather(2 * g + 1, rows1, gs1)
            gwait(rows0, gs0)
            scatter(2 * g, rows0)
            gather(2 * g + 2, rows0, gs0)
            gwait(rows1, gs1)
            scatter(2 * g + 1, rows1)
            return 0
        lax.fori_loop(0, ch // 2 - 1, body, 0)
        gather(ch - 1, rows1, gs1)
        gwait(rows0, gs0)
        scatter(ch - 2, rows0)
        gwait(rows1, gs1)
        scatter(ch - 1, rows1)

    with jax.named_scope("agg_edges"):
        @pl.when(s < NS - 1)
        def _():
            off = 0
            for ch in CH_BIG:
                run_chunk(e_hbm, pl.multiple_of(s * RPT_BIG_E + off, 8), ch)
                off += ch

        @pl.when(s == NS - 1)
        def _():
            off = 0
            for ch in CH_MAIN15:
                run_chunk(e_hbm, (NS - 1) * RPT_BIG_E + off, ch)
                off += ch
            run_chunk(tail_hbm, 0, TAILR)

    with jax.named_scope("agg_wb"):
        plsc.subcore_barrier()

        def wb(nch):
            ds_ = [pltpu.async_copy(
                acc.at[pl.ds(rbase + k * WCH, WCH)],
                out_hbm.at[pl.ds(rbase + k * WCH, WCH)], gs0)
                for k in range(nch)]
            for d in ds_:
                d.wait()

        @pl.when(s < NS - 1)
        def _():
            wb(RPT_BIG // WCH)

        @pl.when(s == NS - 1)
        def _():
            wb(RPT_LAST // WCH)


def _mlp_body(x_ref, agg_ref, w0, b0, w1, b1, w2, b2, out_ref):
    h = x_ref[...] + agg_ref[...]
    for w, b in ((w0, b0), (w1, b1), (w2, b2)):
        h = jnp.dot(h, w[...], preferred_element_type=jnp.float32)
        h = jnp.maximum(h + b[...], 0.0)
    out_ref[...] = h


def _head_body(x_ref, agg_ref, w0, b0, w1, b1, w2, b2, fcw, fcb, out_ref):
    h = x_ref[...] + agg_ref[...]
    for w, b in ((w0, b0), (w1, b1), (w2, b2)):
        h = jnp.dot(h, w[...], preferred_element_type=jnp.float32)
        h = jnp.maximum(h + b[...], 0.0)
    logits = jnp.dot(h, fcw[...], preferred_element_type=jnp.float32) + fcb[...]
    m = jnp.max(logits, axis=1, keepdims=True)
    z = logits - m
    lse = jnp.log(jnp.sum(jnp.exp(z), axis=1, keepdims=True))
    out_ref[...] = z - lse


_ROWS_BLK = 1000
_GRID = N // _ROWS_BLK

_x_spec = pl.BlockSpec((_ROWS_BLK, H), lambda i: (i, 0))
_w_spec = pl.BlockSpec((H, H), lambda i: (0, 0))
_b_spec = pl.BlockSpec((1, H), lambda i: (0, 0))


def _mlp_call(x, agg, w0, b0, w1, b1, w2, b2):
    return pl.pallas_call(
        _mlp_body,
        grid=(_GRID,),
        in_specs=[_x_spec, _x_spec,
                  _w_spec, _b_spec, _w_spec, _b_spec, _w_spec, _b_spec],
        out_specs=pl.BlockSpec((_ROWS_BLK, H), lambda i: (i, 0)),
        out_shape=jax.ShapeDtypeStruct((N, H), jnp.float32),
    )(x, agg, w0, b0, w1, b1, w2, b2)


def _head_call(x, agg, w0, b0, w1, b1, w2, b2, fcw, fcb):
    return pl.pallas_call(
        _head_body,
        grid=(_GRID,),
        in_specs=[_x_spec, _x_spec,
                  _w_spec, _b_spec, _w_spec, _b_spec, _w_spec, _b_spec,
                  pl.BlockSpec((H, C), lambda i: (0, 0)),
                  pl.BlockSpec((1, C), lambda i: (0, 0))],
        out_specs=pl.BlockSpec((_ROWS_BLK, C), lambda i: (i, 0)),
        out_shape=jax.ShapeDtypeStruct((N, C), jnp.float32),
    )(x, agg, w0, b0, w1, b1, w2, b2, fcw, fcb)


def _fold_bn(params, prefix):
    inv_std = 1.0 / jnp.sqrt(1.0 + BN_EPS)
    out = []
    for i in range(3):
        scale = params[f"{prefix}_g{i}"] * inv_std
        out.append(params[f"{prefix}_W{i}"] * scale[None, :])
        out.append((params[f"{prefix}_b{i}"] * scale
                    + params[f"{prefix}_beta{i}"])[None, :])
    return out


def kernel(x, edge_index, edge_attr, params):
    del edge_attr
    e3 = edge_index.astype(jnp.int32).reshape(2, EROWS, EB)
    npadr = TAILR - 20
    pad_dst = N + (jnp.arange(npadr * EB, dtype=jnp.int32) % EB)
    pad_block = jnp.stack([jnp.zeros((npadr * EB,), jnp.int32),
                           pad_dst]).reshape(2, npadr, EB)
    tail = jnp.concatenate([e3[:, EROWS - 20:], pad_block], axis=1)

    c1 = _fold_bn(params, "c1")
    c2 = _fold_bn(params, "c2")
    fcw = params["fc_W"]
    fcb = params["fc_b"][None, :]

    agg1 = _sc_agg(x, e3, tail)
    h1 = _mlp_call(x, agg1, *c1)
    agg2 = _sc_agg(h1, e3, tail)
    return _head_call(h1, agg2, *c2, fcw, fcb)
